# baseline (speedup 1.0000x reference)
import functools

import jax
import jax.numpy as jnp
from jax import lax
from jax.experimental import pallas as pl
from jax.experimental.pallas import tpu as pltpu
from jax.experimental.pallas import tpu_sc as plsc

N_FACTORS = 64
BATCH = 16384
NC = 2
NS = 16
NW = NC * NS
B_PER_W = BATCH // NW
CHUNK = 128
N_CHUNKS = B_PER_W // CHUNK
GROUPS = B_PER_W // 16


def _mf_kernel(u_idx_hbm, i_idx_hbm, uf_hbm, if_hbm, ub_hbm, ib_hbm, gb_hbm,
               out_hbm,
               ui_v, ii_v, urows, irows, ubias, ibias, gb_v, out_v, sem):
    wid = lax.axis_index("s") * NC + lax.axis_index("c")

    pltpu.sync_copy(u_idx_hbm.at[wid], ui_v)
    pltpu.sync_copy(i_idx_hbm.at[wid], ii_v)
    pltpu.sync_copy(gb_hbm, gb_v)

    copies = []
    for j in range(N_CHUNKS):
        sl = pl.ds(j * CHUNK, CHUNK)
        copies.append(pltpu.async_copy(uf_hbm.at[ui_v.at[j]], urows.at[sl], sem))
        copies.append(pltpu.async_copy(if_hbm.at[ii_v.at[j]], irows.at[sl], sem))
        copies.append(pltpu.async_copy(ub_hbm.at[ui_v.at[j]], ubias.at[sl], sem))
        copies.append(pltpu.async_copy(ib_hbm.at[ii_v.at[j]], ibias.at[sl], sem))
    for c in copies:
        c.wait()

    iota16 = lax.iota(jnp.int32, 16)
    zeros16 = jnp.zeros((16,), jnp.int32)
    gb = gb_v[...]

    def group_body(g, carry):
        row = g * 16 + iota16
        acc = gb + plsc.load_gather(ubias, [row])
        acc = acc + plsc.load_gather(ibias, [row])
        for d in range(N_FACTORS):
            dcol = jnp.full((16,), d, jnp.int32)
            uc = plsc.load_gather(urows, [row, dcol])
            ic = plsc.load_gather(irows, [row, dcol])
            acc = acc + uc * ic
        out_v[pl.ds(g * 16, 16)] = acc
        return carry

    lax.fori_loop(0, GROUPS, group_body, 0, unroll=False)

    pltpu.sync_copy(out_v, out_hbm.at[pl.ds(wid * B_PER_W, B_PER_W)])


@jax.jit
def kernel(user_idx, item_idx, user_factors, item_factors, user_biases,
           item_biases, global_bias):
    u_idx = user_idx.astype(jnp.int32).reshape(NW, N_CHUNKS, CHUNK)
    i_idx = item_idx.astype(jnp.int32).reshape(NW, N_CHUNKS, CHUNK)
    gb16 = jnp.broadcast_to(global_bias.astype(jnp.float32), (16,))
    ub1d = user_biases.reshape(-1)
    ib1d = item_biases.reshape(-1)

    mesh = plsc.VectorSubcoreMesh(core_axis_name="c", subcore_axis_name="s")
    run = pl.kernel(
        _mf_kernel,
        mesh=mesh,
        out_type=jax.ShapeDtypeStruct((BATCH,), jnp.float32),
        compiler_params=pltpu.CompilerParams(
            needs_layout_passes=False, use_tc_tiling_on_sc=False),
        scratch_types=[
            pltpu.VMEM((N_CHUNKS, CHUNK), jnp.int32),
            pltpu.VMEM((N_CHUNKS, CHUNK), jnp.int32),
            pltpu.VMEM((B_PER_W, N_FACTORS), jnp.float32),
            pltpu.VMEM((B_PER_W, N_FACTORS), jnp.float32),
            pltpu.VMEM((B_PER_W,), jnp.float32),
            pltpu.VMEM((B_PER_W,), jnp.float32),
            pltpu.VMEM((16,), jnp.float32),
            pltpu.VMEM((B_PER_W,), jnp.float32),
            pltpu.SemaphoreType.DMA,
        ],
    )
    return run(u_idx, i_idx, user_factors, item_factors, ub1d, ib1d, gb16)
